# baseline (speedup 1.0000x reference)
import functools

import jax
import jax.numpy as jnp
from jax import lax
from jax.experimental import pallas as pl
from jax.experimental.pallas import tpu as pltpu
from jax.experimental.pallas import tpu_sc as plsc

_N = 10000
_E = 320000
_D = 128
_OUT = 64
_NC = 2
_NS = 16
_K = 50
_CH = _E // (_NC * _NS * _K)
_RPT = _N // _NS
_RB = 2000

_mesh = plsc.VectorSubcoreMesh(core_axis_name="c", subcore_axis_name="s")


@functools.partial(
    pl.kernel,
    out_type=jax.ShapeDtypeStruct((_NC * _N,), jnp.float32),
    mesh=_mesh,
    scratch_types=[
        pltpu.VMEM((_CH, _K), jnp.int32),
        pltpu.VMEM((128,), jnp.float32),
        pltpu.VMEM((624,), jnp.float32),
        pltpu.VMEM((16,), jnp.float32),
        pltpu.VMEM_SHARED((_N,), jnp.float32)
    ],
)
def _deg_kernel(dst_hbm, ones_hbm, deg_hbm, idx_v, ones_v, stage_v, tail_v,
                acc_sh):
    c = lax.axis_index("c")
    s = lax.axis_index("s")
    wid = c * _NS + s

    pltpu.sync_copy(ones_hbm, ones_v)
    pltpu.sync_copy(dst_hbm.at[pl.ds(wid * _CH, _CH)], idx_v)

    zero = jnp.zeros((16,), jnp.float32)

    def _z(i, carry):
        stage_v[pl.ds(i * 16, 16)] = zero
        return carry

    lax.fori_loop(0, 39, _z, 0)
    tail_v[...] = zero
    pltpu.sync_copy(stage_v, acc_sh.at[pl.ds(s * 624, 624)])

    @pl.when(s == 0)
    def _():
        pltpu.sync_copy(tail_v, acc_sh.at[pl.ds(624 * _NS, 16)])

    plsc.subcore_barrier()

    def _chunk(j, carry):
        pltpu.sync_copy(ones_v.at[pl.ds(0, _K)], acc_sh.at[idx_v.at[j]],
                        add=True)
        return carry

    lax.fori_loop(0, _CH, _chunk, 0)
    plsc.subcore_barrier()

    pltpu.sync_copy(acc_sh.at[pl.ds(s * 624, 624)], stage_v)
    pltpu.sync_copy(stage_v, deg_hbm.at[pl.ds(c * _N + s * 624, 624)])

    @pl.when(s == 0)
    def _():
        pltpu.sync_copy(acc_sh.at[pl.ds(624 * _NS, 16)], tail_v)
        pltpu.sync_copy(tail_v, deg_hbm.at[pl.ds(c * _N + 624 * _NS, 16)])


_CHP = _E // (_NC * _NS * _K)
_GB = 8
_NGB = _CHP // _GB


@functools.partial(
    pl.kernel,
    out_type=jax.ShapeDtypeStruct((_NC, _N, _D), jnp.float32),
    mesh=_mesh,
    scratch_types=[
        pltpu.VMEM((2, _GB, _K), jnp.int32),
        pltpu.VMEM((2, _GB, _K), jnp.int32),
        pltpu.VMEM((_K, _D), jnp.float32),
        pltpu.VMEM((_K, _D), jnp.float32),
        pltpu.VMEM((_K, _D), jnp.float32),
        pltpu.VMEM((_K, _D), jnp.float32),
        pltpu.VMEM((104, _D), jnp.float32),
        pltpu.VMEM_SHARED((_N, _D), jnp.float32),
        pltpu.SemaphoreType.DMA,
        pltpu.SemaphoreType.DMA,
        pltpu.SemaphoreType.DMA,
        pltpu.SemaphoreType.DMA,
        pltpu.SemaphoreType.DMA,
        pltpu.SemaphoreType.DMA,
        pltpu.SemaphoreType.DMA,
        pltpu.SemaphoreType.DMA,
        pltpu.SemaphoreType.DMA,
    ],
)
def _prop_kernel(src_hbm, dst_hbm, h_hbm, out_hbm, srcidx, dstidx, r0, r1,
                 r2, r3, stage_v, acc_sh, g0, g1, g2, g3, s0, s1, s2, s3,
                 isem):
    rows = [r0, r1, r2, r3]
    gsem = [g0, g1, g2, g3]
    ssem = [s0, s1, s2, s3]
    c = lax.axis_index("c")
    s = lax.axis_index("s")
    wid = c * _NS + s
    base = wid * _CHP

    pltpu.sync_copy(src_hbm.at[pl.ds(base, _GB)], srcidx.at[0])
    pltpu.sync_copy(dst_hbm.at[pl.ds(base, _GB)], dstidx.at[0])

    zero = jnp.zeros((16,), jnp.float32)

    def _z(r, carry):
        for q in range(_D // 16):
            stage_v[r, pl.ds(q * 16, 16)] = zero
        return carry

    lax.fori_loop(0, 104, _z, 0)

    def _zc(t, carry):
        pltpu.sync_copy(stage_v, acc_sh.at[pl.ds(s * 624 + t * 104, 104)])
        return carry

    lax.fori_loop(0, 6, _zc, 0)

    @pl.when(s == 0)
    def _():
        pltpu.sync_copy(stage_v.at[pl.ds(0, 16)],
                        acc_sh.at[pl.ds(624 * _NS, 16)])

    pltpu.async_copy(h_hbm.at[srcidx.at[0, 0]], rows[0], gsem[0])
    pltpu.async_copy(h_hbm.at[srcidx.at[0, 1]], rows[1], gsem[1])
    plsc.subcore_barrier()

    def _blk(g, carry):
        h = g % 2
        for q in range(_GB):
            b = q % 4
            b2 = (q + 2) % 4
            if q == 2:
                @pl.when(g < _NGB - 1)
                def _():
                    nb = base + (g + 1) * _GB
                    pltpu.async_copy(src_hbm.at[pl.ds(nb, _GB)],
                                     srcidx.at[1 - h], isem)
                    pltpu.async_copy(dst_hbm.at[pl.ds(nb, _GB)],
                                     dstidx.at[1 - h], isem)
            pltpu.make_async_copy(h_hbm.at[srcidx.at[h, q]], rows[b],
                                  gsem[b]).wait()
            pltpu.async_copy(rows[b], acc_sh.at[dstidx.at[h, q]], ssem[b],
                             add=True)
            if q < 2:
                @pl.when(g > 0)
                def _():
                    pltpu.make_async_copy(rows[b2],
                                          acc_sh.at[dstidx.at[h, 0]],
                                          ssem[b2]).wait()
                    pltpu.async_copy(h_hbm.at[srcidx.at[h, q + 2]], rows[b2],
                                     gsem[b2])

                @pl.when(g == 0)
                def _():
                    pltpu.async_copy(h_hbm.at[srcidx.at[h, q + 2]], rows[b2],
                                     gsem[b2])
            elif q < _GB - 2:
                pltpu.make_async_copy(rows[b2], acc_sh.at[dstidx.at[h, 0]],
                                      ssem[b2]).wait()
                pltpu.async_copy(h_hbm.at[srcidx.at[h, q + 2]], rows[b2],
                                 gsem[b2])
            else:
                pltpu.make_async_copy(rows[b2], acc_sh.at[dstidx.at[h, 0]],
                                      ssem[b2]).wait()
                if q == _GB - 2:
                    @pl.when(g < _NGB - 1)
                    def _():
                        pltpu.make_async_copy(src_hbm.at[pl.ds(base, _GB)],
                                              srcidx.at[1 - h], isem).wait()
                        pltpu.make_async_copy(dst_hbm.at[pl.ds(base, _GB)],
                                              dstidx.at[1 - h], isem).wait()
                        pltpu.async_copy(h_hbm.at[srcidx.at[1 - h, 0]],
                                         rows[b2], gsem[b2])
                else:
                    @pl.when(g < _NGB - 1)
                    def _():
                        pltpu.async_copy(h_hbm.at[srcidx.at[1 - h, 1]],
                                         rows[b2], gsem[b2])
        return carry

    lax.fori_loop(0, _NGB, _blk, 0)

    pltpu.make_async_copy(rows[2], acc_sh.at[dstidx.at[1, 0]], ssem[2]).wait()
    pltpu.make_async_copy(rows[3], acc_sh.at[dstidx.at[1, 0]], ssem[3]).wait()
    plsc.subcore_barrier()

    def _out(t, carry):
        pltpu.sync_copy(acc_sh.at[pl.ds(s * 624 + t * 104, 104)], stage_v)
        pltpu.sync_copy(stage_v,
                        out_hbm.at[c, pl.ds(s * 624 + t * 104, 104)])
        return carry

    lax.fori_loop(0, 6, _out, 0)

    @pl.when(s == 0)
    def _():
        pltpu.sync_copy(acc_sh.at[pl.ds(624 * _NS, 16)],
                        stage_v.at[pl.ds(0, 16)])
        pltpu.sync_copy(stage_v.at[pl.ds(0, 16)],
                        out_hbm.at[c, pl.ds(624 * _NS, 16)])


def _tc1_body(deg_ref, x_ref, w1_ref, dinv_ref, h0p_ref):
    deg = deg_ref[0] + deg_ref[1] + 1.0
    dinv = lax.rsqrt(deg)
    h = jnp.dot(x_ref[...], w1_ref[...], preferred_element_type=jnp.float32)
    dinv_ref[...] = dinv
    h0p_ref[...] = h * dinv


_tc1 = pl.pallas_call(
    _tc1_body,
    grid=(_N // _RB,),
    in_specs=[
        pl.BlockSpec((_NC, _RB, 1), lambda i: (0, i, 0)),
        pl.BlockSpec((_RB, _D), lambda i: (i, 0)),
        pl.BlockSpec((_D, _D), lambda i: (0, 0)),
    ],
    out_specs=[
        pl.BlockSpec((_RB, 1), lambda i: (i, 0)),
        pl.BlockSpec((_RB, _D), lambda i: (i, 0)),
    ],
    out_shape=[
        jax.ShapeDtypeStruct((_N, 1), jnp.float32),
        jax.ShapeDtypeStruct((_N, _D), jnp.float32),
    ],
)


def _tc2_body(dinv_ref, acc_ref, h0p_ref, b1_ref, w23_ref, h1p_ref):
    dinv = dinv_ref[...]
    tot = acc_ref[0] + acc_ref[1] + h0p_ref[...]
    x1 = jnp.maximum(tot * dinv + b1_ref[...], 0.0)
    h1p_ref[...] = jnp.dot(
        x1, w23_ref[...], preferred_element_type=jnp.float32) * dinv


_tc2 = pl.pallas_call(
    _tc2_body,
    grid=(_N // _RB,),
    in_specs=[
        pl.BlockSpec((_RB, 1), lambda i: (i, 0)),
        pl.BlockSpec((_NC, _RB, _D), lambda i: (0, i, 0)),
        pl.BlockSpec((_RB, _D), lambda i: (i, 0)),
        pl.BlockSpec((1, _D), lambda i: (0, 0)),
        pl.BlockSpec((_D, _D), lambda i: (0, 0)),
    ],
    out_specs=pl.BlockSpec((_RB, _D), lambda i: (i, 0)),
    out_shape=jax.ShapeDtypeStruct((_N, _D), jnp.float32),
)


def _tc3_body(dinv_ref, acc_ref, h1p_ref, b23_ref, y_ref):
    dinv = dinv_ref[...]
    tot = acc_ref[0] + acc_ref[1] + h1p_ref[...]
    y_ref[...] = tot * dinv + b23_ref[...]


_tc3 = pl.pallas_call(
    _tc3_body,
    grid=(_N // _RB,),
    in_specs=[
        pl.BlockSpec((_RB, 1), lambda i: (i, 0)),
        pl.BlockSpec((_NC, _RB, _D), lambda i: (0, i, 0)),
        pl.BlockSpec((_RB, _D), lambda i: (i, 0)),
        pl.BlockSpec((1, _D), lambda i: (0, 0)),
    ],
    out_specs=pl.BlockSpec((_RB, _D), lambda i: (i, 0)),
    out_shape=jax.ShapeDtypeStruct((_N, _D), jnp.float32),
)


def kernel(x, edge_index, W1, b1, W2, b2, W3, b3):
    src = edge_index[0].reshape(_E // _K, _K)
    dst = edge_index[1].reshape(_E // _K, _K)
    ones = jnp.ones((128,), jnp.float32)

    deg = _deg_kernel(dst, ones)
    dinv, h0p = _tc1(deg.reshape(_NC, _N, 1), x, W1)
    acc1 = _prop_kernel(src, dst, h0p)

    W23 = jnp.concatenate([W2, W3], axis=1)
    b23 = jnp.concatenate([b2, b3]).reshape(1, _D)
    h1p = _tc2(dinv, acc1, h0p, b1.reshape(1, _D), W23)
    acc2 = _prop_kernel(src, dst, h1p)
    y = _tc3(dinv, acc2, h1p, b23)
    return (y[:, :_OUT], y[:, _OUT:])


# speedup vs baseline: 1.0696x; 1.0696x over previous
import functools

import jax
import jax.numpy as jnp
from jax import lax
from jax.experimental import pallas as pl
from jax.experimental.pallas import tpu as pltpu
from jax.experimental.pallas import tpu_sc as plsc

_N = 10000
_E = 320000
_D = 128
_OUT = 64
_NC = 2
_NS = 16
_K = 125
_CH = _E // (_NC * _NS * _K)
_RPT = _N // _NS
_RB = 2000

_mesh = plsc.VectorSubcoreMesh(core_axis_name="c", subcore_axis_name="s")


@functools.partial(
    pl.kernel,
    out_type=jax.ShapeDtypeStruct((_NC * _N,), jnp.float32),
    mesh=_mesh,
    scratch_types=[
        pltpu.VMEM((_CH, _K), jnp.int32),
        pltpu.VMEM((128,), jnp.float32),
        pltpu.VMEM((624,), jnp.float32),
        pltpu.VMEM((16,), jnp.float32),
        pltpu.VMEM_SHARED((_N,), jnp.float32)
    ],
)
def _deg_kernel(edges_hbm, ones_hbm, deg_hbm, idx_v, ones_v, stage_v,
                tail_v, acc_sh):
    c = lax.axis_index("c")
    s = lax.axis_index("s")
    wid = c * _NS + s

    pltpu.sync_copy(ones_hbm, ones_v)
    pltpu.sync_copy(edges_hbm.at[1, pl.ds(wid * _CH, _CH)], idx_v)

    zero = jnp.zeros((16,), jnp.float32)

    def _z(i, carry):
        stage_v[pl.ds(i * 16, 16)] = zero
        return carry

    lax.fori_loop(0, 39, _z, 0)
    tail_v[...] = zero
    pltpu.sync_copy(stage_v, acc_sh.at[pl.ds(s * 624, 624)])

    @pl.when(s == 0)
    def _():
        pltpu.sync_copy(tail_v, acc_sh.at[pl.ds(624 * _NS, 16)])

    plsc.subcore_barrier()

    def _chunk(j, carry):
        pltpu.sync_copy(ones_v.at[pl.ds(0, _K)], acc_sh.at[idx_v.at[j]],
                        add=True)
        return carry

    lax.fori_loop(0, _CH, _chunk, 0)
    plsc.subcore_barrier()

    pltpu.sync_copy(acc_sh.at[pl.ds(s * 624, 624)], stage_v)
    pltpu.sync_copy(stage_v, deg_hbm.at[pl.ds(c * _N + s * 624, 624)])

    @pl.when(s == 0)
    def _():
        pltpu.sync_copy(acc_sh.at[pl.ds(624 * _NS, 16)], tail_v)
        pltpu.sync_copy(tail_v, deg_hbm.at[pl.ds(c * _N + 624 * _NS, 16)])


_GB = 8
_NGB = _CH // _GB


@functools.partial(
    pl.kernel,
    out_type=jax.ShapeDtypeStruct((_NC, _N, _D), jnp.float32),
    mesh=_mesh,
    scratch_types=[
        pltpu.VMEM((2, _GB, _K), jnp.int32),
        pltpu.VMEM((2, _GB, _K), jnp.int32),
        pltpu.VMEM((_K, _D), jnp.float32),
        pltpu.VMEM((_K, _D), jnp.float32),
        pltpu.VMEM_SHARED((_N, _D), jnp.float32),
        pltpu.SemaphoreType.DMA,
        pltpu.SemaphoreType.DMA,
        pltpu.SemaphoreType.DMA,
        pltpu.SemaphoreType.DMA,
        pltpu.SemaphoreType.DMA,
    ],
)
def _prop_kernel(edges_hbm, h_hbm, out_hbm, srcidx, dstidx, r0, r1,
                 acc_sh, g0, g1, s0, s1, isem):
    rows = [r0, r1]
    gsem = [g0, g1]
    ssem = [s0, s1]
    c = lax.axis_index("c")
    s = lax.axis_index("s")
    wid = c * _NS + s
    base = wid * _CH

    pltpu.sync_copy(edges_hbm.at[0, pl.ds(base, _GB)], srcidx.at[0])
    pltpu.sync_copy(edges_hbm.at[1, pl.ds(base, _GB)], dstidx.at[0])

    zero = jnp.zeros((16,), jnp.float32)

    def _z(r, carry):
        for q in range(_D // 16):
            rows[0][r, pl.ds(q * 16, 16)] = zero
        return carry

    lax.fori_loop(0, _K, _z, 0)

    def _zc(t, carry):
        pltpu.sync_copy(rows[0].at[pl.ds(0, 104)],
                        acc_sh.at[pl.ds(s * 624 + t * 104, 104)])
        return carry

    lax.fori_loop(0, 6, _zc, 0)

    @pl.when(s == 0)
    def _():
        pltpu.sync_copy(rows[0].at[pl.ds(0, 16)],
                        acc_sh.at[pl.ds(624 * _NS, 16)])

    pltpu.async_copy(h_hbm.at[srcidx.at[0, 0]], rows[0], gsem[0])
    plsc.subcore_barrier()

    def _blk(g, carry):
        h = g % 2

        @pl.when(g < _NGB - 1)
        def _():
            nb = base + (g + 1) * _GB
            pltpu.async_copy(edges_hbm.at[0, pl.ds(nb, _GB)],
                             srcidx.at[1 - h], isem)
            pltpu.async_copy(edges_hbm.at[1, pl.ds(nb, _GB)],
                             dstidx.at[1 - h], isem)

        for q in range(_GB):
            b = q % 2
            pltpu.make_async_copy(h_hbm.at[srcidx.at[h, q]], rows[b],
                                  gsem[b]).wait()
            pltpu.async_copy(rows[b], acc_sh.at[dstidx.at[h, q]], ssem[b],
                             add=True)
            if q == 0:
                @pl.when(g > 0)
                def _():
                    pltpu.make_async_copy(rows[1], acc_sh.at[dstidx.at[h, 0]],
                                          ssem[1]).wait()
            else:
                pltpu.make_async_copy(rows[1 - b],
                                      acc_sh.at[dstidx.at[h, q - 1]],
                                      ssem[1 - b]).wait()
            if q < _GB - 1:
                pltpu.async_copy(h_hbm.at[srcidx.at[h, q + 1]], rows[1 - b],
                                 gsem[1 - b])
            else:
                @pl.when(g < _NGB - 1)
                def _():
                    pltpu.make_async_copy(edges_hbm.at[0, pl.ds(base, _GB)],
                                          srcidx.at[1 - h], isem).wait()
                    pltpu.make_async_copy(edges_hbm.at[1, pl.ds(base, _GB)],
                                          dstidx.at[1 - h], isem).wait()
                    pltpu.async_copy(h_hbm.at[srcidx.at[1 - h, 0]],
                                     rows[1 - b], gsem[1 - b])
        return carry

    lax.fori_loop(0, _NGB, _blk, 0)

    pltpu.make_async_copy(rows[1], acc_sh.at[dstidx.at[1, 0]], ssem[1]).wait()
    plsc.subcore_barrier()

    def _out(t, carry):
        pltpu.sync_copy(acc_sh.at[pl.ds(s * 624 + t * 104, 104)],
                        rows[0].at[pl.ds(0, 104)])
        pltpu.sync_copy(rows[0].at[pl.ds(0, 104)],
                        out_hbm.at[c, pl.ds(s * 624 + t * 104, 104)])
        return carry

    lax.fori_loop(0, 6, _out, 0)

    @pl.when(s == 0)
    def _():
        pltpu.sync_copy(acc_sh.at[pl.ds(624 * _NS, 16)],
                        rows[0].at[pl.ds(0, 16)])
        pltpu.sync_copy(rows[0].at[pl.ds(0, 16)],
                        out_hbm.at[c, pl.ds(624 * _NS, 16)])


def _tc1_body(deg_ref, x_ref, w1_ref, dinv_ref, h0p_ref):
    deg = deg_ref[0] + deg_ref[1] + 1.0
    dinv = lax.rsqrt(deg)
    h = jnp.dot(x_ref[...], w1_ref[...], preferred_element_type=jnp.float32)
    dinv_ref[...] = dinv
    h0p_ref[...] = h * dinv


_tc1 = pl.pallas_call(
    _tc1_body,
    grid=(_N // _RB,),
    in_specs=[
        pl.BlockSpec((_NC, _RB, 1), lambda i: (0, i, 0)),
        pl.BlockSpec((_RB, _D), lambda i: (i, 0)),
        pl.BlockSpec((_D, _D), lambda i: (0, 0)),
    ],
    out_specs=[
        pl.BlockSpec((_RB, 1), lambda i: (i, 0)),
        pl.BlockSpec((_RB, _D), lambda i: (i, 0)),
    ],
    out_shape=[
        jax.ShapeDtypeStruct((_N, 1), jnp.float32),
        jax.ShapeDtypeStruct((_N, _D), jnp.float32),
    ],
)


def _tc2_body(dinv_ref, acc_ref, h0p_ref, b1_ref, w23_ref, h1p_ref):
    dinv = dinv_ref[...]
    tot = acc_ref[0] + acc_ref[1] + h0p_ref[...]
    x1 = jnp.maximum(tot * dinv + b1_ref[...], 0.0)
    h1p_ref[...] = jnp.dot(
        x1, w23_ref[...], preferred_element_type=jnp.float32) * dinv


_tc2 = pl.pallas_call(
    _tc2_body,
    grid=(_N // _RB,),
    in_specs=[
        pl.BlockSpec((_RB, 1), lambda i: (i, 0)),
        pl.BlockSpec((_NC, _RB, _D), lambda i: (0, i, 0)),
        pl.BlockSpec((_RB, _D), lambda i: (i, 0)),
        pl.BlockSpec((1, _D), lambda i: (0, 0)),
        pl.BlockSpec((_D, _D), lambda i: (0, 0)),
    ],
    out_specs=pl.BlockSpec((_RB, _D), lambda i: (i, 0)),
    out_shape=jax.ShapeDtypeStruct((_N, _D), jnp.float32),
)


def _tc3_body(dinv_ref, acc_ref, h1p_ref, b23_ref, mu_ref, ls_ref):
    dinv = dinv_ref[...]
    tot = acc_ref[0] + acc_ref[1] + h1p_ref[...]
    y = tot * dinv + b23_ref[...]
    mu_ref[...] = y[:, :_OUT]
    ls_ref[...] = y[:, _OUT:]


_tc3 = pl.pallas_call(
    _tc3_body,
    grid=(_N // _RB,),
    in_specs=[
        pl.BlockSpec((_RB, 1), lambda i: (i, 0)),
        pl.BlockSpec((_NC, _RB, _D), lambda i: (0, i, 0)),
        pl.BlockSpec((_RB, _D), lambda i: (i, 0)),
        pl.BlockSpec((1, _D), lambda i: (0, 0)),
    ],
    out_specs=[
        pl.BlockSpec((_RB, _OUT), lambda i: (i, 0)),
        pl.BlockSpec((_RB, _OUT), lambda i: (i, 0)),
    ],
    out_shape=[
        jax.ShapeDtypeStruct((_N, _OUT), jnp.float32),
        jax.ShapeDtypeStruct((_N, _OUT), jnp.float32),
    ],
)


def kernel(x, edge_index, W1, b1, W2, b2, W3, b3):
    edges = edge_index.reshape(2, _E // _K, _K)
    ones = jnp.ones((128,), jnp.float32)

    deg = _deg_kernel(edges, ones)
    dinv, h0p = _tc1(deg.reshape(_NC, _N, 1), x, W1)
    acc1 = _prop_kernel(edges, h0p)

    W23 = jnp.concatenate([W2, W3], axis=1)
    b23 = jnp.concatenate([b2, b3]).reshape(1, _D)
    h1p = _tc2(dinv, acc1, h0p, b1.reshape(1, _D), W23)
    acc2 = _prop_kernel(edges, h1p)
    mu, logstd = _tc3(dinv, acc2, h1p, b23)
    return (mu, logstd)


# speedup vs baseline: 1.0835x; 1.0129x over previous
import functools

import jax
import jax.numpy as jnp
from jax import lax
from jax.experimental import pallas as pl
from jax.experimental.pallas import tpu as pltpu
from jax.experimental.pallas import tpu_sc as plsc

_N = 10000
_E = 320000
_D = 128
_OUT = 64
_NC = 2
_NS = 16
_K = 125
_CH = _E // (_NC * _NS * _K)
_RPT = _N // _NS
_RB = 2000

_mesh = plsc.VectorSubcoreMesh(core_axis_name="c", subcore_axis_name="s")


@functools.partial(
    pl.kernel,
    out_type=jax.ShapeDtypeStruct((_NC * _N,), jnp.float32),
    mesh=_mesh,
    scratch_types=[
        pltpu.VMEM((_CH, _K), jnp.int32),
        pltpu.VMEM((128,), jnp.float32),
        pltpu.VMEM((624,), jnp.float32),
        pltpu.VMEM((16,), jnp.float32),
        pltpu.VMEM_SHARED((_N,), jnp.float32)
    ],
)
def _deg_kernel(edges_hbm, ones_hbm, deg_hbm, idx_v, ones_v, stage_v,
                tail_v, acc_sh):
    c = lax.axis_index("c")
    s = lax.axis_index("s")
    wid = c * _NS + s

    pltpu.sync_copy(ones_hbm, ones_v)
    pltpu.sync_copy(edges_hbm.at[1, pl.ds(wid * _CH, _CH)], idx_v)

    zero = jnp.zeros((16,), jnp.float32)

    def _z(i, carry):
        stage_v[pl.ds(i * 16, 16)] = zero
        return carry

    lax.fori_loop(0, 39, _z, 0)
    tail_v[...] = zero
    pltpu.sync_copy(stage_v, acc_sh.at[pl.ds(s * 624, 624)])

    @pl.when(s == 0)
    def _():
        pltpu.sync_copy(tail_v, acc_sh.at[pl.ds(624 * _NS, 16)])

    plsc.subcore_barrier()

    def _chunk(j, carry):
        pltpu.sync_copy(ones_v.at[pl.ds(0, _K)], acc_sh.at[idx_v.at[j]],
                        add=True)
        return carry

    lax.fori_loop(0, _CH, _chunk, 0)
    plsc.subcore_barrier()

    pltpu.sync_copy(acc_sh.at[pl.ds(s * 624, 624)], stage_v)
    pltpu.sync_copy(stage_v, deg_hbm.at[pl.ds(c * _N + s * 624, 624)])

    @pl.when(s == 0)
    def _():
        pltpu.sync_copy(acc_sh.at[pl.ds(624 * _NS, 16)], tail_v)
        pltpu.sync_copy(tail_v, deg_hbm.at[pl.ds(c * _N + 624 * _NS, 16)])


_GB = 8
_NGB = _CH // _GB


@functools.partial(
    pl.kernel,
    out_type=jax.ShapeDtypeStruct((_NC, _N, _D), jnp.float32),
    mesh=_mesh,
    scratch_types=[
        pltpu.VMEM((2, _GB, _K), jnp.int32),
        pltpu.VMEM((2, _GB, _K), jnp.int32),
        pltpu.VMEM((_K, _D), jnp.float32),
        pltpu.VMEM((_K, _D), jnp.float32),
        pltpu.VMEM_SHARED((_N, _D), jnp.float32),
        pltpu.SemaphoreType.DMA,
        pltpu.SemaphoreType.DMA,
        pltpu.SemaphoreType.DMA,
        pltpu.SemaphoreType.DMA,
        pltpu.SemaphoreType.DMA,
    ],
)
def _prop_kernel(edges_hbm, h_hbm, out_hbm, srcidx, dstidx, r0, r1,
                 acc_sh, g0, g1, s0, s1, isem):
    rows = [r0, r1]
    gsem = [g0, g1]
    ssem = [s0, s1]
    c = lax.axis_index("c")
    s = lax.axis_index("s")
    wid = c * _NS + s
    base = wid * _CH

    pltpu.sync_copy(edges_hbm.at[0, pl.ds(base, _GB)], srcidx.at[0])
    pltpu.sync_copy(edges_hbm.at[1, pl.ds(base, _GB)], dstidx.at[0])

    zero = jnp.zeros((16,), jnp.float32)

    def _z(r, carry):
        for q in range(_D // 16):
            rows[0][r, pl.ds(q * 16, 16)] = zero
        return carry

    lax.fori_loop(0, _K, _z, 0)

    def _zc(t, carry):
        pltpu.async_copy(rows[0].at[pl.ds(0, 104)],
                         acc_sh.at[pl.ds(s * 624 + t * 104, 104)], isem)
        return carry

    lax.fori_loop(0, 6, _zc, 0)

    @pl.when(s == 0)
    def _():
        pltpu.async_copy(rows[0].at[pl.ds(0, 16)],
                         acc_sh.at[pl.ds(624 * _NS, 16)], isem)

    def _zw(t, carry):
        pltpu.make_async_copy(rows[0].at[pl.ds(0, 104)],
                              acc_sh.at[pl.ds(s * 624, 104)], isem).wait()
        return carry

    lax.fori_loop(0, 6, _zw, 0)

    @pl.when(s == 0)
    def _():
        pltpu.make_async_copy(rows[0].at[pl.ds(0, 16)],
                              acc_sh.at[pl.ds(624 * _NS, 16)], isem).wait()

    pltpu.async_copy(h_hbm.at[srcidx.at[0, 0]], rows[0], gsem[0])
    plsc.subcore_barrier()

    def _blk(g, carry):
        h = g % 2

        @pl.when(g < _NGB - 1)
        def _():
            nb = base + (g + 1) * _GB
            pltpu.async_copy(edges_hbm.at[0, pl.ds(nb, _GB)],
                             srcidx.at[1 - h], isem)
            pltpu.async_copy(edges_hbm.at[1, pl.ds(nb, _GB)],
                             dstidx.at[1 - h], isem)

        for q in range(_GB):
            b = q % 2
            pltpu.make_async_copy(h_hbm.at[srcidx.at[h, q]], rows[b],
                                  gsem[b]).wait()
            pltpu.async_copy(rows[b], acc_sh.at[dstidx.at[h, q]], ssem[b],
                             add=True)
            if q == 0:
                @pl.when(g > 0)
                def _():
                    pltpu.make_async_copy(rows[1], acc_sh.at[dstidx.at[h, 0]],
                                          ssem[1]).wait()
            else:
                pltpu.make_async_copy(rows[1 - b],
                                      acc_sh.at[dstidx.at[h, q - 1]],
                                      ssem[1 - b]).wait()
            if q < _GB - 1:
                pltpu.async_copy(h_hbm.at[srcidx.at[h, q + 1]], rows[1 - b],
                                 gsem[1 - b])
            else:
                @pl.when(g < _NGB - 1)
                def _():
                    pltpu.make_async_copy(edges_hbm.at[0, pl.ds(base, _GB)],
                                          srcidx.at[1 - h], isem).wait()
                    pltpu.make_async_copy(edges_hbm.at[1, pl.ds(base, _GB)],
                                          dstidx.at[1 - h], isem).wait()
                    pltpu.async_copy(h_hbm.at[srcidx.at[1 - h, 0]],
                                     rows[1 - b], gsem[1 - b])
        return carry

    lax.fori_loop(0, _NGB, _blk, 0)

    pltpu.make_async_copy(rows[1], acc_sh.at[dstidx.at[1, 0]], ssem[1]).wait()
    plsc.subcore_barrier()

    for t in range(6):
        b = t % 2
        if t >= 2:
            pltpu.make_async_copy(
                rows[b].at[pl.ds(0, 104)],
                out_hbm.at[c, pl.ds(s * 624, 104)], ssem[b]).wait()
        pltpu.sync_copy(acc_sh.at[pl.ds(s * 624 + t * 104, 104)],
                        rows[b].at[pl.ds(0, 104)])
        pltpu.async_copy(rows[b].at[pl.ds(0, 104)],
                         out_hbm.at[c, pl.ds(s * 624 + t * 104, 104)],
                         ssem[b])
    for b in range(2):
        pltpu.make_async_copy(rows[b].at[pl.ds(0, 104)],
                              out_hbm.at[c, pl.ds(s * 624, 104)],
                              ssem[b]).wait()

    @pl.when(s == 0)
    def _():
        pltpu.sync_copy(acc_sh.at[pl.ds(624 * _NS, 16)],
                        rows[0].at[pl.ds(0, 16)])
        pltpu.sync_copy(rows[0].at[pl.ds(0, 16)],
                        out_hbm.at[c, pl.ds(624 * _NS, 16)])


def _tc1_body(deg_ref, x_ref, w1_ref, dinv_ref, h0p_ref):
    deg = deg_ref[0] + deg_ref[1] + 1.0
    dinv = lax.rsqrt(deg)
    h = jnp.dot(x_ref[...], w1_ref[...], preferred_element_type=jnp.float32)
    dinv_ref[...] = dinv
    h0p_ref[...] = h * dinv


_tc1 = pl.pallas_call(
    _tc1_body,
    grid=(_N // _RB,),
    in_specs=[
        pl.BlockSpec((_NC, _RB, 1), lambda i: (0, i, 0)),
        pl.BlockSpec((_RB, _D), lambda i: (i, 0)),
        pl.BlockSpec((_D, _D), lambda i: (0, 0)),
    ],
    out_specs=[
        pl.BlockSpec((_RB, 1), lambda i: (i, 0)),
        pl.BlockSpec((_RB, _D), lambda i: (i, 0)),
    ],
    out_shape=[
        jax.ShapeDtypeStruct((_N, 1), jnp.float32),
        jax.ShapeDtypeStruct((_N, _D), jnp.float32),
    ],
)


def _tc2_body(dinv_ref, acc_ref, h0p_ref, b1_ref, w23_ref, h1p_ref):
    dinv = dinv_ref[...]
    tot = acc_ref[0] + acc_ref[1] + h0p_ref[...]
    x1 = jnp.maximum(tot * dinv + b1_ref[...], 0.0)
    h1p_ref[...] = jnp.dot(
        x1, w23_ref[...], preferred_element_type=jnp.float32) * dinv


_tc2 = pl.pallas_call(
    _tc2_body,
    grid=(_N // _RB,),
    in_specs=[
        pl.BlockSpec((_RB, 1), lambda i: (i, 0)),
        pl.BlockSpec((_NC, _RB, _D), lambda i: (0, i, 0)),
        pl.BlockSpec((_RB, _D), lambda i: (i, 0)),
        pl.BlockSpec((1, _D), lambda i: (0, 0)),
        pl.BlockSpec((_D, _D), lambda i: (0, 0)),
    ],
    out_specs=pl.BlockSpec((_RB, _D), lambda i: (i, 0)),
    out_shape=jax.ShapeDtypeStruct((_N, _D), jnp.float32),
)


def _tc3_body(dinv_ref, acc_ref, h1p_ref, b23_ref, mu_ref, ls_ref):
    dinv = dinv_ref[...]
    tot = acc_ref[0] + acc_ref[1] + h1p_ref[...]
    y = tot * dinv + b23_ref[...]
    mu_ref[...] = y[:, :_OUT]
    ls_ref[...] = y[:, _OUT:]


_tc3 = pl.pallas_call(
    _tc3_body,
    grid=(_N // _RB,),
    in_specs=[
        pl.BlockSpec((_RB, 1), lambda i: (i, 0)),
        pl.BlockSpec((_NC, _RB, _D), lambda i: (0, i, 0)),
        pl.BlockSpec((_RB, _D), lambda i: (i, 0)),
        pl.BlockSpec((1, _D), lambda i: (0, 0)),
    ],
    out_specs=[
        pl.BlockSpec((_RB, _OUT), lambda i: (i, 0)),
        pl.BlockSpec((_RB, _OUT), lambda i: (i, 0)),
    ],
    out_shape=[
        jax.ShapeDtypeStruct((_N, _OUT), jnp.float32),
        jax.ShapeDtypeStruct((_N, _OUT), jnp.float32),
    ],
)


def kernel(x, edge_index, W1, b1, W2, b2, W3, b3):
    edges = edge_index.reshape(2, _E // _K, _K)
    ones = jnp.ones((128,), jnp.float32)

    deg = _deg_kernel(edges, ones)
    dinv, h0p = _tc1(deg.reshape(_NC, _N, 1), x, W1)
    acc1 = _prop_kernel(edges, h0p)

    W23 = jnp.concatenate([W2, W3], axis=1)
    b23 = jnp.concatenate([b2, b3]).reshape(1, _D)
    h1p = _tc2(dinv, acc1, h0p, b1.reshape(1, _D), W23)
    acc2 = _prop_kernel(edges, h1p)
    mu, logstd = _tc3(dinv, acc2, h1p, b23)
    return (mu, logstd)
